# baseline (speedup 1.0000x reference)
import functools

import jax
import jax.numpy as jnp
from jax import lax
from jax.experimental import pallas as pl
from jax.experimental.pallas import tpu as pltpu
from jax.experimental.pallas import tpu_sc as plsc

BATCH = 16384
EMB = 128
LANES = 16
NUM_CORES = 2
NUM_SUBCORES = 16
NUM_WORKERS = NUM_CORES * NUM_SUBCORES
BPW = BATCH // NUM_WORKERS
CHUNK = 64
NCHUNK = BPW // CHUNK
NBUF = 3
DCHUNKS = EMB // LANES
GROUPS = CHUNK // LANES


def _compute_chunk(rows1, rowsr, rows2, accs, out_v, ck):
  def row_body(i, carry):
    acc = (rows1[i, pl.ds(0, LANES)]
           * rowsr[i, pl.ds(0, LANES)]
           * rows2[i, pl.ds(0, LANES)])
    for j in range(1, DCHUNKS):
      acc = acc + (rows1[i, pl.ds(j * LANES, LANES)]
                   * rowsr[i, pl.ds(j * LANES, LANES)]
                   * rows2[i, pl.ds(j * LANES, LANES)])
    accs[pl.ds(i * LANES, LANES)] = acc
    return carry

  lax.fori_loop(0, CHUNK, row_body, 0)

  def group(g, carry):
    iota = lax.iota(jnp.int32, LANES)
    rowbase = (g * LANES + iota) * LANES
    ssum = plsc.load_gather(accs, [rowbase + iota])
    for d in range(1, LANES):
      col = jnp.bitwise_and(iota + d, LANES - 1)
      ssum = ssum + plsc.load_gather(accs, [rowbase + col])
    out_v[pl.ds(ck * CHUNK + g * LANES, LANES)] = ssum
    return carry

  lax.fori_loop(0, GROUPS, group, 0)


def _score_body(e1i_hbm, ri_hbm, e2i_hbm, ent_hbm, rel_hbm, out_hbm,
                idx1, idxr, idx2,
                rows1, rowsr, rows2, sems,
                accs, out_v):
  wid = lax.axis_index("s") * NUM_CORES + lax.axis_index("c")
  base = wid * BPW

  pltpu.sync_copy(e1i_hbm.at[pl.ds(wid * NCHUNK, NCHUNK)], idx1)
  pltpu.sync_copy(ri_hbm.at[pl.ds(wid * NCHUNK, NCHUNK)], idxr)
  pltpu.sync_copy(e2i_hbm.at[pl.ds(wid * NCHUNK, NCHUNK)], idx2)

  def fire(ck):
    buf = ck % NBUF
    return (
        pltpu.async_copy(ent_hbm.at[idx1.at[ck]], rows1[buf], sems[buf]),
        pltpu.async_copy(rel_hbm.at[idxr.at[ck]], rowsr[buf], sems[buf]),
        pltpu.async_copy(ent_hbm.at[idx2.at[ck]], rows2[buf], sems[buf]),
    )

  pending = [fire(ck) for ck in range(NBUF)]
  for ck in range(NCHUNK):
    buf = ck % NBUF
    cur = pending.pop(0)
    for h in cur:
      h.wait()
    _compute_chunk(rows1[buf], rowsr[buf], rows2[buf], accs, out_v, ck)
    if ck + NBUF < NCHUNK:
      pending.append(fire(ck + NBUF))

  pltpu.sync_copy(out_v, out_hbm.at[pl.ds(base, BPW)])


@functools.partial(
    pl.kernel,
    out_type=jax.ShapeDtypeStruct((BATCH,), jnp.float32),
    mesh=plsc.VectorSubcoreMesh(core_axis_name="c", subcore_axis_name="s"),
    scratch_types=[
        pltpu.VMEM((NCHUNK, CHUNK), jnp.int32),
        pltpu.VMEM((NCHUNK, CHUNK), jnp.int32),
        pltpu.VMEM((NCHUNK, CHUNK), jnp.int32),
        *([pltpu.VMEM((CHUNK, EMB), jnp.float32)] * (3 * NBUF)),
        pltpu.VMEM((CHUNK * LANES,), jnp.float32),
        pltpu.VMEM((BPW,), jnp.float32),
        *([pltpu.SemaphoreType.DMA] * NBUF),
    ],
    compiler_params=pltpu.CompilerParams(needs_layout_passes=False),
)
def _score_kernel(e1i, ri, e2i, ent, rel, out,
                  idx1, idxr, idx2, *rest):
  rows = rest[:3 * NBUF]
  accs, out_v = rest[3 * NBUF], rest[3 * NBUF + 1]
  sems = rest[3 * NBUF + 2:]
  _score_body(e1i, ri, e2i, ent, rel, out,
              idx1, idxr, idx2,
              rows[0::3], rows[1::3], rows[2::3], sems,
              accs, out_v)


@jax.jit
def kernel(x, entity_emb, relation_emb):
  e1i = x[:, 0].reshape(NUM_WORKERS * NCHUNK, CHUNK)
  ri = x[:, 1].reshape(NUM_WORKERS * NCHUNK, CHUNK)
  e2i = x[:, 2].reshape(NUM_WORKERS * NCHUNK, CHUNK)
  return _score_kernel(e1i, ri, e2i, entity_emb, relation_emb)


# speedup vs baseline: 1.0006x; 1.0006x over previous
import functools

import jax
import jax.numpy as jnp
from jax import lax
from jax.experimental import pallas as pl
from jax.experimental.pallas import tpu as pltpu
from jax.experimental.pallas import tpu_sc as plsc

BATCH = 16384
EMB = 128
LANES = 16
NUM_CORES = 2
NUM_SUBCORES = 16
NUM_WORKERS = NUM_CORES * NUM_SUBCORES
BPW = BATCH // NUM_WORKERS
CHUNK = 64
NCHUNK = BPW // CHUNK
NBUF = 4
DCHUNKS = EMB // LANES
GROUPS = CHUNK // LANES


def _compute_chunk(rows1, rowsr, rows2, accs, out_v, ck):
  def row_body(i, carry):
    acc = (rows1[i, pl.ds(0, LANES)]
           * rowsr[i, pl.ds(0, LANES)]
           * rows2[i, pl.ds(0, LANES)])
    for j in range(1, DCHUNKS):
      acc = acc + (rows1[i, pl.ds(j * LANES, LANES)]
                   * rowsr[i, pl.ds(j * LANES, LANES)]
                   * rows2[i, pl.ds(j * LANES, LANES)])
    accs[pl.ds(i * LANES, LANES)] = acc
    return carry

  lax.fori_loop(0, CHUNK, row_body, 0)

  def group(g, carry):
    iota = lax.iota(jnp.int32, LANES)
    rowbase = (g * LANES + iota) * LANES
    ssum = plsc.load_gather(accs, [rowbase + iota])
    for d in range(1, LANES):
      col = jnp.bitwise_and(iota + d, LANES - 1)
      ssum = ssum + plsc.load_gather(accs, [rowbase + col])
    out_v[pl.ds(ck * CHUNK + g * LANES, LANES)] = ssum
    return carry

  lax.fori_loop(0, GROUPS, group, 0)


def _score_body(e1i_hbm, ri_hbm, e2i_hbm, ent_hbm, rel_hbm, out_hbm,
                idx1, idxr, idx2,
                rows1, rowsr, rows2, sems,
                accs, out_v):
  wid = lax.axis_index("s") * NUM_CORES + lax.axis_index("c")
  base = wid * BPW

  pltpu.sync_copy(e1i_hbm.at[pl.ds(wid * NCHUNK, NCHUNK)], idx1)
  pltpu.sync_copy(ri_hbm.at[pl.ds(wid * NCHUNK, NCHUNK)], idxr)
  pltpu.sync_copy(e2i_hbm.at[pl.ds(wid * NCHUNK, NCHUNK)], idx2)

  def fire(ck):
    buf = ck % NBUF
    return (
        pltpu.async_copy(ent_hbm.at[idx1.at[ck]], rows1[buf], sems[buf]),
        pltpu.async_copy(rel_hbm.at[idxr.at[ck]], rowsr[buf], sems[buf]),
        pltpu.async_copy(ent_hbm.at[idx2.at[ck]], rows2[buf], sems[buf]),
    )

  pending = [fire(ck) for ck in range(NBUF - 1)]
  for ck in range(NCHUNK):
    buf = ck % NBUF
    cur = pending.pop(0)
    for h in cur:
      h.wait()
    if ck + NBUF - 1 < NCHUNK:
      pending.append(fire(ck + NBUF - 1))
    _compute_chunk(rows1[buf], rowsr[buf], rows2[buf], accs, out_v, ck)

  pltpu.sync_copy(out_v, out_hbm.at[pl.ds(base, BPW)])


@functools.partial(
    pl.kernel,
    out_type=jax.ShapeDtypeStruct((BATCH,), jnp.float32),
    mesh=plsc.VectorSubcoreMesh(core_axis_name="c", subcore_axis_name="s"),
    scratch_types=[
        pltpu.VMEM((NCHUNK, CHUNK), jnp.int32),
        pltpu.VMEM((NCHUNK, CHUNK), jnp.int32),
        pltpu.VMEM((NCHUNK, CHUNK), jnp.int32),
        *([pltpu.VMEM((CHUNK, EMB), jnp.float32)] * (3 * NBUF)),
        pltpu.VMEM((CHUNK * LANES,), jnp.float32),
        pltpu.VMEM((BPW,), jnp.float32),
        *([pltpu.SemaphoreType.DMA] * NBUF),
    ],
    compiler_params=pltpu.CompilerParams(needs_layout_passes=False),
)
def _score_kernel(e1i, ri, e2i, ent, rel, out,
                  idx1, idxr, idx2, *rest):
  rows = rest[:3 * NBUF]
  accs, out_v = rest[3 * NBUF], rest[3 * NBUF + 1]
  sems = rest[3 * NBUF + 2:]
  _score_body(e1i, ri, e2i, ent, rel, out,
              idx1, idxr, idx2,
              rows[0::3], rows[1::3], rows[2::3], sems,
              accs, out_v)


@jax.jit
def kernel(x, entity_emb, relation_emb):
  e1i = x[:, 0].reshape(NUM_WORKERS * NCHUNK, CHUNK)
  ri = x[:, 1].reshape(NUM_WORKERS * NCHUNK, CHUNK)
  e2i = x[:, 2].reshape(NUM_WORKERS * NCHUNK, CHUNK)
  return _score_kernel(e1i, ri, e2i, entity_emb, relation_emb)
